# baseline (speedup 1.0000x reference)
import functools

import jax
import jax.numpy as jnp
from jax import lax
from jax.experimental import pallas as pl
from jax.experimental.pallas import tpu as pltpu
from jax.experimental.pallas import tpu_sc as plsc

N = 100000
E = 1600000
D = 4
BIG = 32
HID = 2
G = 128

NPAD = 102400
EPAD = 1638400
W = 128
TILE_E = 1024
TILE_N = 1024
NSUB = 16
STRIPE = NPAD // NSUB

_SC_MESH = functools.partial(
    plsc.VectorSubcoreMesh, core_axis_name="c", subcore_axis_name="s")


def _stats_body(x_ref, o_ref):
    @pl.when(pl.program_id(0) == 0)
    def _():
        o_ref[...] = jnp.zeros_like(o_ref)

    xb = x_ref[...]
    o_ref[:, 0:1] += jnp.sum(xb, axis=1, keepdims=True)
    o_ref[:, 1:2] += jnp.sum(xb * xb, axis=1, keepdims=True)


def _stats(xT):
    return pl.pallas_call(
        _stats_body,
        grid=(NPAD // TILE_N,),
        in_specs=[pl.BlockSpec((D, TILE_N), lambda i: (0, i))],
        out_specs=pl.BlockSpec((D, 2), lambda i: (0, 0)),
        out_shape=jax.ShapeDtypeStruct((D, 2), jnp.float32),
    )(xT)


def _sc_gather(tables, dstw, srcw, kw):
    nt = len(tables)

    @functools.partial(
        pl.kernel,
        out_type=jax.ShapeDtypeStruct((2 * nt, EPAD), jnp.float32),
        mesh=_SC_MESH(),
        scratch_types=[pltpu.VMEM_SHARED((NPAD,), jnp.float32)
                       for _ in range(nt)]
        + [pltpu.VMEM((STRIPE,), jnp.float32)],
    )
    def k(*refs):
        t_hbm = refs[:nt]
        di_hbm = refs[nt]
        si_hbm = refs[nt + 1]
        o_hbm = refs[nt + 2]
        stab = refs[nt + 3:2 * nt + 3]
        vbuf = refs[-1]

        sid = lax.axis_index("s")
        sl = pl.ds(sid * STRIPE, STRIPE)
        for c in range(nt):
            pltpu.sync_copy(t_hbm[c].at[sl], vbuf)
            pltpu.sync_copy(vbuf, stab[c].at[sl])
        plsc.subcore_barrier()

        def body(*bufs):
            di = bufs[:kw]
            si = bufs[kw:2 * kw]
            ov = bufs[2 * kw:]
            for j in range(kw):
                for c in range(nt):
                    pltpu.sync_copy(stab[c].at[di[j].at[0]], ov[j].at[c])
                    pltpu.sync_copy(stab[c].at[si[j].at[0]], ov[j].at[nt + c])

        idx_spec = lambda j: pl.BlockSpec(
            (1, W), lambda i, j=j: (i * kw + j, 0))
        out_spec = lambda j: pl.BlockSpec(
            (2 * nt, W), lambda i, j=j: (0, i * kw + j))
        pltpu.emit_pipeline(
            body,
            grid=(EPAD // (kw * W),),
            in_specs=[idx_spec(j) for j in range(kw)] * 2,
            out_specs=[out_spec(j) for j in range(kw)],
            core_axis_name=("c", "s"),
            dimension_semantics=(pltpu.PARALLEL,),
        )(*([di_hbm] * kw), *([si_hbm] * kw), *([o_hbm] * kw))

    return k(*tables, dstw, srcw)


def _sc_scatter(m, dstw, npl, kw):
    @functools.partial(
        pl.kernel,
        out_type=jax.ShapeDtypeStruct((2 * npl * NPAD,), jnp.float32),
        mesh=_SC_MESH(),
        scratch_types=[pltpu.VMEM_SHARED((NPAD,), jnp.float32)
                       for _ in range(npl)]
        + [pltpu.VMEM((STRIPE,), jnp.float32)],
    )
    def k(m_hbm, di_hbm, o_hbm, *scr):
        acc = scr[:npl]
        vbuf = scr[-1]
        cid = lax.axis_index("c")
        sid = lax.axis_index("s")
        sl = pl.ds(sid * STRIPE, STRIPE)

        @pl.loop(0, STRIPE, step=16)
        def _(i):
            vbuf[pl.ds(i, 16)] = jnp.zeros((16,), jnp.float32)

        for c in range(npl):
            pltpu.sync_copy(vbuf, acc[c].at[sl])
        plsc.subcore_barrier()

        def body(*bufs):
            mv = bufs[:kw]
            di = bufs[kw:]
            for j in range(kw):
                for c in range(npl):
                    pltpu.sync_copy(mv[j].at[c], acc[c].at[di[j].at[0]],
                                    add=True)

        m_spec = lambda j: pl.BlockSpec(
            (npl, W), lambda i, j=j: (0, i * kw + j))
        idx_spec = lambda j: pl.BlockSpec(
            (1, W), lambda i, j=j: (i * kw + j, 0))
        pltpu.emit_pipeline(
            body,
            grid=(EPAD // (kw * W),),
            in_specs=[m_spec(j) for j in range(kw)]
            + [idx_spec(j) for j in range(kw)],
            out_specs=[],
            core_axis_name=("c", "s"),
            dimension_semantics=(pltpu.PARALLEL,),
        )(*([m_hbm] * kw), *([di_hbm] * kw))
        plsc.subcore_barrier()
        for c in range(npl):
            pltpu.sync_copy(acc[c].at[sl], vbuf)
            pltpu.sync_copy(
                vbuf,
                o_hbm.at[pl.ds((cid * npl + c) * NPAD + sid * STRIPE,
                               STRIPE)])

    return k(m, dstw).reshape(2, npl, NPAD)


def _mlp_body(ni, final_relu, x_ref, waT_ref, wbT_ref, c1_ref,
              w2T_ref, b2_ref, w3T_ref, b3_ref, o_ref):
    x = x_ref[...]
    waT = waT_ref[...]
    wbT = wbT_ref[...]
    acc = None
    for c in range(ni):
        t = (waT[:, c:c + 1] * x[c:c + 1, :]
             + wbT[:, c:c + 1] * x[ni + c:ni + c + 1, :])
        acc = t if acc is None else acc + t
    h = jnp.maximum(acc + c1_ref[...], 0.0)
    h = jnp.maximum(
        jnp.dot(w2T_ref[...], h, precision="highest") + b2_ref[...], 0.0)
    h = jnp.dot(w3T_ref[...], h, precision="highest") + b3_ref[...]
    if final_relu:
        h = jnp.maximum(h, 0.0)
    o_ref[...] = h


def _edge_mlp(xe, waT, wbT, c1, w2T, b2, w3T, b3, final_relu):
    ni = waT.shape[1]
    no = w3T.shape[0]
    full = lambda r, c: pl.BlockSpec((r, c), lambda i: (0, 0))
    return pl.pallas_call(
        functools.partial(_mlp_body, ni, final_relu),
        grid=(EPAD // TILE_E,),
        in_specs=[pl.BlockSpec((2 * ni, TILE_E), lambda i: (0, i)),
                  full(BIG, ni), full(BIG, ni), full(BIG, 1),
                  full(BIG, BIG), full(BIG, 1),
                  full(no, BIG), full(no, 1)],
        out_specs=pl.BlockSpec((no, TILE_E), lambda i: (0, i)),
        out_shape=jax.ShapeDtypeStruct((no, EPAD), jnp.float32),
    )(xe, waT, wbT, c1, w2T, b2, w3T, b3)


def _combine_body(p_ref, h_ref, cnt_ref):
    p = p_ref[...]
    s = p[0] + p[1]
    cnt = s[2:3, :]
    denom = jnp.maximum(cnt, 1.0)
    h_ref[...] = s[0:2, :] / denom
    cnt_ref[...] = cnt


def _combine(penc):
    return pl.pallas_call(
        _combine_body,
        grid=(NPAD // TILE_N,),
        in_specs=[pl.BlockSpec((2, 3, TILE_N), lambda i: (0, 0, i))],
        out_specs=(pl.BlockSpec((2, TILE_N), lambda i: (0, i)),
                   pl.BlockSpec((1, TILE_N), lambda i: (0, i))),
        out_shape=(jax.ShapeDtypeStruct((2, NPAD), jnp.float32),
                   jax.ShapeDtypeStruct((1, NPAD), jnp.float32)),
    )(penc)


def _final_body(p_ref, cnt_ref, x_ref, b_ref, out_ref, emd_ref,
                acc_sum, acc_cnt):
    i = pl.program_id(0)

    @pl.when(i == 0)
    def _():
        acc_sum[...] = jnp.zeros_like(acc_sum)
        acc_cnt[...] = jnp.zeros_like(acc_cnt)

    p = p_ref[...]
    s = p[0] + p[1]
    denom = jnp.maximum(cnt_ref[...], 1.0)
    out = s / denom
    out_ref[...] = out
    d = out - x_ref[...]
    diff = jnp.sum(d * d, axis=0, keepdims=True)
    onehot = (lax.broadcasted_iota(jnp.int32, (G, 1), 0) ==
              b_ref[...]).astype(jnp.float32)
    acc_sum[...] += jnp.sum(onehot * diff, axis=1, keepdims=True)
    acc_cnt[...] += jnp.sum(onehot, axis=1, keepdims=True)

    @pl.when(i == pl.num_programs(0) - 1)
    def _():
        emd_ref[...] = acc_sum[...] / jnp.maximum(acc_cnt[...], 1.0)


def _final(pdec, cnt, xT, batch2):
    return pl.pallas_call(
        _final_body,
        grid=(NPAD // TILE_N,),
        in_specs=[pl.BlockSpec((2, D, TILE_N), lambda i: (0, 0, i)),
                  pl.BlockSpec((1, TILE_N), lambda i: (0, i)),
                  pl.BlockSpec((D, TILE_N), lambda i: (0, i)),
                  pl.BlockSpec((1, TILE_N), lambda i: (0, i))],
        out_specs=(pl.BlockSpec((D, TILE_N), lambda i: (0, i)),
                   pl.BlockSpec((G, 1), lambda i: (0, 0))),
        out_shape=(jax.ShapeDtypeStruct((D, NPAD), jnp.float32),
                   jax.ShapeDtypeStruct((G, 1), jnp.float32)),
        scratch_shapes=[pltpu.VMEM((G, 1), jnp.float32),
                        pltpu.VMEM((G, 1), jnp.float32)],
    )(pdec, cnt, xT, batch2)


def kernel(x, edge_index, batch, bn_gamma, bn_beta,
           enc_w1, enc_b1, enc_w2, enc_b2, enc_w3, enc_b3,
           dec_w1, dec_b1, dec_w2, dec_b2, dec_w3, dec_b3):
    f32 = jnp.float32
    xT = jnp.pad(x.T, ((0, 0), (0, NPAD - N)))
    ei = jnp.pad(edge_index.astype(jnp.int32), ((0, 0), (0, EPAD - E)),
                 constant_values=NPAD - 1)
    srcw = ei[0].reshape(EPAD // W, W)
    dstw = ei[1].reshape(EPAD // W, W)
    batch2 = jnp.pad(batch.astype(jnp.int32), (0, NPAD - N),
                     constant_values=-1).reshape(1, NPAD)

    st = _stats(xT)
    mean = st[:, 0] / N
    var = st[:, 1] / N - mean * mean
    scale = bn_gamma / jnp.sqrt(var + 1e-5)
    shift = bn_beta - mean * scale

    e_waT = (scale[:, None] * (enc_w1[:D] - enc_w1[D:])).T
    e_wbT = (scale[:, None] * enc_w1[D:]).T
    e_c1 = (enc_b1 + shift @ enc_w1[:D]).reshape(BIG, 1)
    e_w3T = jnp.concatenate([enc_w3.T, jnp.zeros((1, BIG), f32)])
    e_b3 = jnp.concatenate([enc_b3, jnp.ones((1,), f32)]).reshape(3, 1)

    d_waT = (dec_w1[:HID] - dec_w1[HID:]).T
    d_wbT = dec_w1[HID:].T
    d_c1 = dec_b1.reshape(BIG, 1)

    xe = _sc_gather([xT[0], xT[1], xT[2], xT[3]], dstw, srcw, 2)
    m1 = _edge_mlp(xe, e_waT, e_wbT, e_c1, enc_w2.T,
                   enc_b2.reshape(BIG, 1), e_w3T, e_b3, final_relu=True)
    penc = _sc_scatter(m1, dstw, 3, 4)
    h2d, cnt = _combine(penc)

    xd = _sc_gather([h2d[0], h2d[1]], dstw, srcw, 4)
    m2 = _edge_mlp(xd, d_waT, d_wbT, d_c1, dec_w2.T,
                   dec_b2.reshape(BIG, 1), dec_w3.T,
                   dec_b3.reshape(D, 1), final_relu=False)
    pdec = _sc_scatter(m2, dstw, 4, 4)

    outT, emd = _final(pdec, cnt, xT, batch2)
    return outT[:, :N].T, emd.reshape(G)


# speedup vs baseline: 1.2474x; 1.2474x over previous
import functools

import jax
import jax.numpy as jnp
from jax import lax
from jax.experimental import pallas as pl
from jax.experimental.pallas import tpu as pltpu
from jax.experimental.pallas import tpu_sc as plsc

N = 100000
E = 1600000
D = 4
BIG = 32
HID = 2
G = 128

NPAD = 102400
EPAD = 1638400
W = 128
TILE_E = 1024
TILE_N = 1024
NSUB = 16
STRIPE = NPAD // NSUB

_SC_MESH = functools.partial(
    plsc.VectorSubcoreMesh, core_axis_name="c", subcore_axis_name="s")


def _stats_body(x_ref, o_ref):
    @pl.when(pl.program_id(0) == 0)
    def _():
        o_ref[...] = jnp.zeros_like(o_ref)

    xb = x_ref[...]
    o_ref[:, 0:1] += jnp.sum(xb, axis=1, keepdims=True)
    o_ref[:, 1:2] += jnp.sum(xb * xb, axis=1, keepdims=True)


def _stats(xT):
    return pl.pallas_call(
        _stats_body,
        grid=(NPAD // TILE_N,),
        in_specs=[pl.BlockSpec((D, TILE_N), lambda i: (0, i))],
        out_specs=pl.BlockSpec((D, 2), lambda i: (0, 0)),
        out_shape=jax.ShapeDtypeStruct((D, 2), jnp.float32),
    )(xT)


def _sc_gather(tables, dstw, srcw, kw):
    nt = len(tables)

    @functools.partial(
        pl.kernel,
        out_type=jax.ShapeDtypeStruct((2 * nt, EPAD), jnp.float32),
        mesh=_SC_MESH(),
        scratch_types=[pltpu.VMEM_SHARED((NPAD,), jnp.float32)
                       for _ in range(nt)]
        + [pltpu.VMEM((STRIPE,), jnp.float32)],
    )
    def k(*refs):
        t_hbm = refs[:nt]
        di_hbm = refs[nt]
        si_hbm = refs[nt + 1]
        o_hbm = refs[nt + 2]
        stab = refs[nt + 3:2 * nt + 3]
        vbuf = refs[-1]

        sid = lax.axis_index("s")
        sl = pl.ds(sid * STRIPE, STRIPE)
        for c in range(nt):
            pltpu.sync_copy(t_hbm[c].at[sl], vbuf)
            pltpu.sync_copy(vbuf, stab[c].at[sl])
        plsc.subcore_barrier()

        def body(*bufs):
            di = bufs[:kw]
            si = bufs[kw:2 * kw]
            ov = bufs[2 * kw:]
            for j in range(kw):
                for c in range(nt):
                    pltpu.sync_copy(stab[c].at[di[j].at[0]], ov[j].at[c])
                    pltpu.sync_copy(stab[c].at[si[j].at[0]], ov[j].at[nt + c])

        idx_spec = lambda j: pl.BlockSpec(
            (1, W), lambda i, j=j: (i * kw + j, 0))
        out_spec = lambda j: pl.BlockSpec(
            (2 * nt, W), lambda i, j=j: (0, i * kw + j))
        pltpu.emit_pipeline(
            body,
            grid=(EPAD // (kw * W),),
            in_specs=[idx_spec(j) for j in range(kw)] * 2,
            out_specs=[out_spec(j) for j in range(kw)],
            core_axis_name=("c", "s"),
            dimension_semantics=(pltpu.PARALLEL,),
        )(*([di_hbm] * kw), *([si_hbm] * kw), *([o_hbm] * kw))

    return k(*tables, dstw, srcw)


def _sc_scatter(m, dstw, npl, kw):
    @functools.partial(
        pl.kernel,
        out_type=jax.ShapeDtypeStruct((2 * npl * NPAD,), jnp.float32),
        mesh=_SC_MESH(),
        scratch_types=[pltpu.VMEM_SHARED((NPAD,), jnp.float32)
                       for _ in range(npl)]
        + [pltpu.VMEM((STRIPE,), jnp.float32)],
    )
    def k(m_hbm, di_hbm, o_hbm, *scr):
        acc = scr[:npl]
        vbuf = scr[-1]
        cid = lax.axis_index("c")
        sid = lax.axis_index("s")
        sl = pl.ds(sid * STRIPE, STRIPE)

        @pl.loop(0, STRIPE, step=16)
        def _(i):
            vbuf[pl.ds(i, 16)] = jnp.zeros((16,), jnp.float32)

        for c in range(npl):
            pltpu.sync_copy(vbuf, acc[c].at[sl])
        plsc.subcore_barrier()

        def body(*bufs):
            mv = bufs[:kw]
            di = bufs[kw:]
            for j in range(kw):
                for c in range(npl):
                    pltpu.sync_copy(mv[j].at[c], acc[c].at[di[j].at[0]],
                                    add=True)

        m_spec = lambda j: pl.BlockSpec(
            (npl, W), lambda i, j=j: (0, i * kw + j))
        idx_spec = lambda j: pl.BlockSpec(
            (1, W), lambda i, j=j: (i * kw + j, 0))
        pltpu.emit_pipeline(
            body,
            grid=(EPAD // (kw * W),),
            in_specs=[m_spec(j) for j in range(kw)]
            + [idx_spec(j) for j in range(kw)],
            out_specs=[],
            core_axis_name=("c", "s"),
            dimension_semantics=(pltpu.PARALLEL,),
        )(*([m_hbm] * kw), *([di_hbm] * kw))
        plsc.subcore_barrier()
        for c in range(npl):
            pltpu.sync_copy(acc[c].at[sl], vbuf)
            pltpu.sync_copy(
                vbuf,
                o_hbm.at[pl.ds((cid * npl + c) * NPAD + sid * STRIPE,
                               STRIPE)])

    return k(m, dstw).reshape(2, npl, NPAD)


def _mlp_body(ni, final_relu, x_ref, waT_ref, wbT_ref, c1_ref,
              w2T_ref, b2_ref, w3T_ref, b3_ref, o_ref):
    x = x_ref[...]
    waT = waT_ref[...]
    wbT = wbT_ref[...]
    acc = None
    for c in range(ni):
        t = (waT[:, c:c + 1] * x[c:c + 1, :]
             + wbT[:, c:c + 1] * x[ni + c:ni + c + 1, :])
        acc = t if acc is None else acc + t
    h = jnp.maximum(acc + c1_ref[...], 0.0)
    h = jnp.maximum(
        jnp.dot(w2T_ref[...], h, precision="highest") + b2_ref[...], 0.0)
    h = jnp.dot(w3T_ref[...], h, precision="highest") + b3_ref[...]
    if final_relu:
        h = jnp.maximum(h, 0.0)
    o_ref[...] = h


def _edge_mlp(xe, waT, wbT, c1, w2T, b2, w3T, b3, final_relu):
    ni = waT.shape[1]
    no = w3T.shape[0]
    full = lambda r, c: pl.BlockSpec((r, c), lambda i: (0, 0))
    return pl.pallas_call(
        functools.partial(_mlp_body, ni, final_relu),
        grid=(EPAD // TILE_E,),
        in_specs=[pl.BlockSpec((2 * ni, TILE_E), lambda i: (0, i)),
                  full(BIG, ni), full(BIG, ni), full(BIG, 1),
                  full(BIG, BIG), full(BIG, 1),
                  full(no, BIG), full(no, 1)],
        out_specs=pl.BlockSpec((no, TILE_E), lambda i: (0, i)),
        out_shape=jax.ShapeDtypeStruct((no, EPAD), jnp.float32),
    )(xe, waT, wbT, c1, w2T, b2, w3T, b3)


def _combine_body(p_ref, h_ref, cnt_ref):
    p = p_ref[...]
    s = p[0] + p[1]
    cnt = s[2:3, :]
    denom = jnp.maximum(cnt, 1.0)
    h_ref[...] = s[0:2, :] / denom
    cnt_ref[...] = cnt


def _combine(penc):
    return pl.pallas_call(
        _combine_body,
        grid=(NPAD // TILE_N,),
        in_specs=[pl.BlockSpec((2, 3, TILE_N), lambda i: (0, 0, i))],
        out_specs=(pl.BlockSpec((2, TILE_N), lambda i: (0, i)),
                   pl.BlockSpec((1, TILE_N), lambda i: (0, i))),
        out_shape=(jax.ShapeDtypeStruct((2, NPAD), jnp.float32),
                   jax.ShapeDtypeStruct((1, NPAD), jnp.float32)),
    )(penc)


def _final_body(p_ref, cnt_ref, x_ref, b_ref, out_ref, emd_ref,
                acc_sum, acc_cnt):
    i = pl.program_id(0)

    @pl.when(i == 0)
    def _():
        acc_sum[...] = jnp.zeros_like(acc_sum)
        acc_cnt[...] = jnp.zeros_like(acc_cnt)

    p = p_ref[...]
    s = p[0] + p[1]
    denom = jnp.maximum(cnt_ref[...], 1.0)
    out = s / denom
    out_ref[...] = out
    d = out - x_ref[...]
    diff = jnp.sum(d * d, axis=0, keepdims=True)
    onehot = (lax.broadcasted_iota(jnp.int32, (G, 1), 0) ==
              b_ref[...]).astype(jnp.float32)
    acc_sum[...] += jnp.sum(onehot * diff, axis=1, keepdims=True)
    acc_cnt[...] += jnp.sum(onehot, axis=1, keepdims=True)

    @pl.when(i == pl.num_programs(0) - 1)
    def _():
        emd_ref[...] = acc_sum[...] / jnp.maximum(acc_cnt[...], 1.0)


def _final(pdec, cnt, xT, batch2):
    return pl.pallas_call(
        _final_body,
        grid=(NPAD // TILE_N,),
        in_specs=[pl.BlockSpec((2, D, TILE_N), lambda i: (0, 0, i)),
                  pl.BlockSpec((1, TILE_N), lambda i: (0, i)),
                  pl.BlockSpec((D, TILE_N), lambda i: (0, i)),
                  pl.BlockSpec((1, TILE_N), lambda i: (0, i))],
        out_specs=(pl.BlockSpec((D, TILE_N), lambda i: (0, i)),
                   pl.BlockSpec((G, 1), lambda i: (0, 0))),
        out_shape=(jax.ShapeDtypeStruct((D, NPAD), jnp.float32),
                   jax.ShapeDtypeStruct((G, 1), jnp.float32)),
        scratch_shapes=[pltpu.VMEM((G, 1), jnp.float32),
                        pltpu.VMEM((G, 1), jnp.float32)],
    )(pdec, cnt, xT, batch2)


def kernel(x, edge_index, batch, bn_gamma, bn_beta,
           enc_w1, enc_b1, enc_w2, enc_b2, enc_w3, enc_b3,
           dec_w1, dec_b1, dec_w2, dec_b2, dec_w3, dec_b3):
    f32 = jnp.float32
    xT = jnp.pad(x.T, ((0, 0), (0, NPAD - N)))
    ei = jnp.pad(edge_index.astype(jnp.int32), ((0, 0), (0, EPAD - E)),
                 constant_values=NPAD - 1)
    srcw = ei[0].reshape(EPAD // W, W)
    dstw = ei[1].reshape(EPAD // W, W)
    batch2 = jnp.pad(batch.astype(jnp.int32), (0, NPAD - N),
                     constant_values=-1).reshape(1, NPAD)

    st = _stats(xT)
    mean = st[:, 0] / N
    var = st[:, 1] / N - mean * mean
    scale = bn_gamma / jnp.sqrt(var + 1e-5)
    shift = bn_beta - mean * scale

    e_waT = (scale[:, None] * (enc_w1[:D] - enc_w1[D:])).T
    e_wbT = (scale[:, None] * enc_w1[D:]).T
    e_c1 = (enc_b1 + shift @ enc_w1[:D]).reshape(BIG, 1)
    e_w3T = jnp.concatenate([enc_w3.T, jnp.zeros((1, BIG), f32)])
    e_b3 = jnp.concatenate([enc_b3, jnp.ones((1,), f32)]).reshape(3, 1)

    d_waT = (dec_w1[:HID] - dec_w1[HID:]).T
    d_wbT = dec_w1[HID:].T
    d_c1 = dec_b1.reshape(BIG, 1)

    xe = _sc_gather([xT[0], xT[1], xT[2], xT[3]], dstw, srcw, 1)
    m1 = _edge_mlp(xe, e_waT, e_wbT, e_c1, enc_w2.T,
                   enc_b2.reshape(BIG, 1), e_w3T, e_b3, final_relu=True)
    penc = _sc_scatter(m1, dstw, 3, 1)
    h2d, cnt = _combine(penc)

    xd = _sc_gather([h2d[0], h2d[1]], dstw, srcw, 1)
    m2 = _edge_mlp(xd, d_waT, d_wbT, d_c1, dec_w2.T,
                   dec_b2.reshape(BIG, 1), dec_w3.T,
                   dec_b3.reshape(D, 1), final_relu=False)
    pdec = _sc_scatter(m2, dstw, 4, 1)

    outT, emd = _final(pdec, cnt, xT, batch2)
    return outT[:, :N].T, emd.reshape(G)
